# baseline (speedup 1.0000x reference)
import functools

import jax
import jax.numpy as jnp
from jax import lax
from jax.experimental import pallas as pl
from jax.experimental.pallas import tpu as pltpu
from jax.experimental.pallas import tpu_sc as plsc

ALPHA = 0.2
LANES = 16
NSUB = 16
NSC = 2
NW = NSC * NSUB
BLK = 128
FW = 128
CHB = 16


def _sc_attn(P, SUB, NP, E, EPT, nblk):
  mesh = plsc.VectorSubcoreMesh(core_axis_name="c", subcore_axis_name="s")
  NPB = NP // BLK
  NRS = SUB * NPB
  NCH = NRS // 8

  @functools.partial(
      pl.kernel,
      out_type=(
          jax.ShapeDtypeStruct((NW, P, nblk, SUB, BLK), jnp.float32),
          jax.ShapeDtypeStruct((NSC, P, NRS, BLK), jnp.float32),
      ),
      mesh=mesh,
      compiler_params=pltpu.CompilerParams(needs_layout_passes=False),
      scratch_types=[
          pltpu.VMEM((nblk, BLK), jnp.int32),
          pltpu.VMEM((nblk, BLK), jnp.int32),
          pltpu.VMEM((SUB * NP,), jnp.float32),
          pltpu.VMEM((SUB * NP,), jnp.float32),
          pltpu.VMEM((NRS, BLK), jnp.float32),
          pltpu.VMEM((nblk, SUB, BLK), jnp.float32),
          pltpu.VMEM((SUB, NPB), jnp.int32),
          pltpu.VMEM((8, BLK), jnp.float32),
          pltpu.VMEM_SHARED((NRS, BLK), jnp.float32),
      ],
  )
  def k(src_hbm, dst_hbm, elT_hbm, erT_hbm, e_hbm, rsum_hbm,
        src_c, dst_c, el_c, er_c, rs, e_all, rsidx, zrs, rs_sh):
    cid = lax.axis_index("c")
    sid = lax.axis_index("s")
    wid = cid * NSUB + sid
    base = wid * EPT

    pltpu.sync_copy(src_hbm.at[wid], src_c)
    pltpu.sync_copy(dst_hbm.at[wid], dst_c)

    @pl.loop(0, 8)
    def _(r):
      for q in range(BLK // LANES):
        zrs[r, pl.ds(LANES * q, LANES)] = jnp.zeros((LANES,), jnp.float32)

    @pl.loop(0, NPB // LANES)
    def _(i):
      for u in range(SUB):
        rsidx[u, pl.ds(LANES * i, LANES)] = (
            u * NPB + LANES * i + lax.iota(jnp.int32, 16))

    @pl.loop(0, P)
    def _(p):
      plsc.subcore_barrier()
      for z in range((NCH + NSUB - 1) // NSUB):
        ch = sid + z * NSUB
        @pl.when(ch < NCH)
        def _():
          pltpu.sync_copy(zrs, rs_sh.at[pl.ds(ch * 8, 8)])
      plsc.subcore_barrier()

      for u in range(SUB):
        pltpu.sync_copy(elT_hbm.at[p * SUB + u], el_c.at[pl.ds(u * NP, NP)])
        pltpu.sync_copy(erT_hbm.at[p * SUB + u], er_c.at[pl.ds(u * NP, NP)])

      @pl.loop(0, NRS)
      def _(r):
        for q in range(BLK // LANES):
          rs[r, pl.ds(LANES * q, LANES)] = jnp.zeros((LANES,), jnp.float32)

      @pl.loop(0, nblk)
      def _(b):
        for i in range(BLK // LANES):
          sl = pl.ds(LANES * i, LANES)
          s16 = src_c[b, sl]
          d16 = dst_c[b, sl]
          gid = base + b * BLK + LANES * i + lax.iota(jnp.int32, 16)
          valid = gid < E
          srow = lax.shift_right_logical(s16, 7)
          slane = jnp.bitwise_and(s16, 127)
          for u in range(SUB):
            el16 = plsc.load_gather(el_c, [s16 + u * NP])
            er16 = plsc.load_gather(er_c, [d16 + u * NP])
            t = el16 + er16
            e16 = jnp.exp(jnp.where(t > 0, t, ALPHA * t))
            e16 = jnp.where(valid, e16, 0.0)
            plsc.addupdate_scatter(rs, [u * NPB + srow, slane], e16)
            e_all[b, u, sl] = e16

      pltpu.sync_copy(e_all, e_hbm.at[wid, p])

      for u in range(SUB):
        pltpu.sync_copy(rs.at[pl.ds(u * NPB, NPB)],
                        rs_sh.at[rsidx.at[u]], add=True)
      plsc.subcore_barrier()

      for z in range((NCH + NSUB - 1) // NSUB):
        ch = sid + z * NSUB
        @pl.when(ch < NCH)
        def _():
          pltpu.sync_copy(rs_sh.at[pl.ds(ch * 8, 8)],
                          rsum_hbm.at[cid, p, pl.ds(ch * 8, 8)])

  return k


def _sc_scatter(P, SUB, NP, EPT, nblk):
  mesh = plsc.VectorSubcoreMesh(core_axis_name="c", subcore_axis_name="s")
  SEGW = FW // SUB
  RPT = NP // NSUB
  NCHK = nblk // CHB
  K = CHB // 2

  @functools.partial(
      pl.kernel,
      out_type=jax.ShapeDtypeStruct((NSC, P, NP, FW), jnp.float32),
      mesh=mesh,
      compiler_params=pltpu.CompilerParams(needs_layout_passes=False),
      scratch_types=[
          pltpu.VMEM((CHB, BLK), jnp.int32),
          pltpu.VMEM((CHB, BLK), jnp.int32),
          pltpu.VMEM((CHB, SUB, BLK), jnp.float32),
          pltpu.VMEM((BLK, FW), jnp.float32),
          pltpu.VMEM((BLK, FW), jnp.float32),
          pltpu.VMEM((1, BLK), jnp.int32),
          pltpu.VMEM((1, BLK), jnp.int32),
          pltpu.VMEM((1, BLK), jnp.int32),
          pltpu.VMEM((1, BLK), jnp.int32),
          pltpu.VMEM((32, FW), jnp.float32),
          pltpu.SemaphoreType.DMA,
          pltpu.SemaphoreType.DMA,
          pltpu.SemaphoreType.DMA,
          pltpu.SemaphoreType.DMA,
          pltpu.SemaphoreType.DMA,
          pltpu.VMEM_SHARED((NP, FW), jnp.float32),
      ],
  )
  def k(src_hbm, dst_hbm, e_hbm, hpack_hbm, hsum_hbm,
        src_cc, dst_cc, e_c, hrows_a, hrows_b, gidx_a, gidx_b, sidx_a,
        sidx_b, zbuf, sem_ga, sem_gb, sem_sa, sem_sb, sem_z, acc_sh):
    cid = lax.axis_index("c")
    sid = lax.axis_index("s")
    wid = cid * NSUB + sid
    row0 = sid * RPT
    dummy = hpack_hbm.at[pl.ds(0, BLK)]

    @pl.loop(0, 32)
    def _(r):
      for q in range(FW // LANES):
        zbuf[r, pl.ds(LANES * q, LANES)] = jnp.zeros((LANES,), jnp.float32)

    def mkidx(bb, p, gx, sx):
      for i in range(BLK // LANES):
        sl = pl.ds(LANES * i, LANES)
        gx[0, sl] = dst_cc[bb, sl] + p * NP
        sx[0, sl] = src_cc[bb, sl]

    def scale(hr, bb):
      for j in range(BLK // LANES):
        e16s = [e_c[bb, u, pl.ds(LANES * j, LANES)] for u in range(SUB)]
        for rr in range(LANES):
          r = LANES * j + rr
          for u in range(SUB):
            ev = jnp.broadcast_to(e16s[u][rr], (LANES,))
            for q in range(SEGW // LANES):
              qs = pl.ds(u * SEGW + LANES * q, LANES)
              hr[r, qs] = hr[r, qs] * ev

    @pl.loop(0, P)
    def _(p):
      plsc.subcore_barrier()
      zs = [pltpu.async_copy(zbuf, acc_sh.at[pl.ds(row0 + z * 32, 32)],
                             sem_z) for z in range(RPT // 32)]
      for z in zs:
        z.wait()
      plsc.subcore_barrier()

      @pl.loop(0, NCHK)
      def _(c):
        pltpu.sync_copy(src_hbm.at[wid, pl.ds(c * CHB, CHB)], src_cc)
        pltpu.sync_copy(dst_hbm.at[wid, pl.ds(c * CHB, CHB)], dst_cc)
        pltpu.sync_copy(e_hbm.at[wid, p, pl.ds(c * CHB, CHB)], e_c)

        mkidx(0, p, gidx_a, sidx_a)
        pltpu.async_copy(hpack_hbm.at[gidx_a.at[0]], hrows_a, sem_ga)

        @pl.loop(0, K)
        def _(kk):
          @pl.when(kk > 0)
          def _():
            pltpu.make_async_copy(dummy, hrows_b, sem_sb).wait()
          mkidx(2 * kk + 1, p, gidx_b, sidx_b)
          gb = pltpu.async_copy(hpack_hbm.at[gidx_b.at[0]], hrows_b, sem_gb)

          pltpu.make_async_copy(dummy, hrows_a, sem_ga).wait()
          scale(hrows_a, 2 * kk)
          sa = pltpu.async_copy(hrows_a, acc_sh.at[sidx_a.at[0]], sem_sa,
                                add=True)

          gb.wait()
          scale(hrows_b, 2 * kk + 1)
          pltpu.async_copy(hrows_b, acc_sh.at[sidx_b.at[0]], sem_sb,
                           add=True)

          sa.wait()
          @pl.when(kk < K - 1)
          def _():
            mkidx(2 * kk + 2, p, gidx_a, sidx_a)
            pltpu.async_copy(hpack_hbm.at[gidx_a.at[0]], hrows_a, sem_ga)

        pltpu.make_async_copy(dummy, hrows_b, sem_sb).wait()

      plsc.subcore_barrier()
      pltpu.sync_copy(acc_sh.at[pl.ds(row0, RPT)],
                      hsum_hbm.at[cid, p, pl.ds(row0, RPT)])

  return k


def _sc_edge_stage2(NP, E, EPT, nblk, NCLASS):
  mesh = plsc.VectorSubcoreMesh(core_axis_name="c", subcore_axis_name="s")
  NPK = NP // 8
  RPT = NPK // NSUB
  NPB = NP // BLK
  NCH = NPB // 8

  @functools.partial(
      pl.kernel,
      out_type=(
          jax.ShapeDtypeStruct((NSC, NPK, FW), jnp.float32),
          jax.ShapeDtypeStruct((NSC, NPB, BLK), jnp.float32),
      ),
      mesh=mesh,
      compiler_params=pltpu.CompilerParams(needs_layout_passes=False),
      scratch_types=[
          pltpu.VMEM((nblk, BLK), jnp.int32),
          pltpu.VMEM((nblk, BLK), jnp.int32),
          pltpu.VMEM((NP,), jnp.float32),
          pltpu.VMEM((NP,), jnp.float32),
          pltpu.VMEM((NPB, BLK), jnp.float32),
          pltpu.VMEM((BLK, FW), jnp.float32),
          pltpu.VMEM((BLK, FW), jnp.float32),
          pltpu.VMEM((1, BLK), jnp.int32),
          pltpu.VMEM((1, BLK), jnp.int32),
          pltpu.VMEM((1, BLK), jnp.int32),
          pltpu.VMEM((1, NPB), jnp.int32),
          pltpu.VMEM((1, BLK), jnp.float32),
          pltpu.VMEM((16, FW), jnp.float32),
          pltpu.VMEM((8, BLK), jnp.float32),
          pltpu.VMEM_SHARED((NPK, FW), jnp.float32),
          pltpu.VMEM_SHARED((NPB, BLK), jnp.float32),
      ],
  )
  def k(src_hbm, dst_hbm, el_hbm, er_hbm, g_hbm, osum_hbm, rsum_hbm,
        src_c, dst_c, el_c, er_c, rs, hrows, srows, gidx, sidx, soff, rsidx,
        e_blk, zbuf, zrs, acc_sh, rs_sh):
    cid = lax.axis_index("c")
    sid = lax.axis_index("s")
    wid = cid * NSUB + sid
    base = wid * EPT
    row0 = sid * RPT

    pltpu.sync_copy(src_hbm.at[wid], src_c)
    pltpu.sync_copy(dst_hbm.at[wid], dst_c)
    pltpu.sync_copy(el_hbm.at[0], el_c)
    pltpu.sync_copy(er_hbm.at[0], er_c)

    @pl.loop(0, 16)
    def _(r):
      for q in range(FW // LANES):
        zbuf[r, pl.ds(LANES * q, LANES)] = jnp.zeros((LANES,), jnp.float32)

    @pl.loop(0, 8)
    def _(r):
      for q in range(BLK // LANES):
        zrs[r, pl.ds(LANES * q, LANES)] = jnp.zeros((LANES,), jnp.float32)

    @pl.loop(0, BLK)
    def _(r):
      for q in range(FW // LANES):
        srows[r, pl.ds(LANES * q, LANES)] = jnp.zeros((LANES,), jnp.float32)

    @pl.loop(0, NPB // LANES)
    def _(i):
      rsidx[0, pl.ds(LANES * i, LANES)] = (
          LANES * i + lax.iota(jnp.int32, 16))

    @pl.loop(0, NPB)
    def _(r):
      for q in range(BLK // LANES):
        rs[r, pl.ds(LANES * q, LANES)] = jnp.zeros((LANES,), jnp.float32)

    for z in range(RPT // 16):
      pltpu.sync_copy(zbuf, acc_sh.at[pl.ds(row0 + z * 16, 16)])

    @pl.when(sid < NCH)
    def _():
      pltpu.sync_copy(zrs, rs_sh.at[pl.ds(sid * 8, 8)])
    plsc.subcore_barrier()

    @pl.loop(0, nblk)
    def _(b):
      for i in range(BLK // LANES):
        sl = pl.ds(LANES * i, LANES)
        s16 = src_c[b, sl]
        d16 = dst_c[b, sl]
        gid = base + b * BLK + LANES * i + lax.iota(jnp.int32, 16)
        el16 = plsc.load_gather(el_c, [s16])
        er16 = plsc.load_gather(er_c, [d16])
        t = el16 + er16
        e16 = jnp.exp(jnp.where(t > 0, t, ALPHA * t))
        e16 = jnp.where(gid < E, e16, 0.0)
        plsc.addupdate_scatter(
            rs, [lax.shift_right_logical(s16, 7), jnp.bitwise_and(s16, 127)],
            e16)
        e_blk[0, sl] = e16
        gidx[0, sl] = d16
        sidx[0, sl] = lax.shift_right_logical(s16, 3)
        soff[0, sl] = jnp.bitwise_and(s16, 7) * LANES
      pltpu.sync_copy(g_hbm.at[gidx.at[0]], hrows)

      @pl.loop(0, BLK // LANES)
      def _(j):
        e16 = e_blk[0, pl.ds(LANES * j, LANES)]
        o16 = soff[0, pl.ds(LANES * j, LANES)]
        for rr in range(LANES):
          r = LANES * j + rr
          v = hrows[r, pl.ds(0, LANES)] * e16[rr]
          srows[r, pl.ds(o16[rr], LANES)] = v

      pltpu.sync_copy(srows, acc_sh.at[sidx.at[0]], add=True)

      @pl.loop(0, BLK // LANES)
      def _(j):
        o16 = soff[0, pl.ds(LANES * j, LANES)]
        for rr in range(LANES):
          srows[LANES * j + rr, pl.ds(o16[rr], LANES)] = jnp.zeros(
              (LANES,), jnp.float32)

    pltpu.sync_copy(rs, rs_sh.at[rsidx.at[0]], add=True)
    plsc.subcore_barrier()

    pltpu.sync_copy(acc_sh.at[pl.ds(row0, RPT)],
                    osum_hbm.at[cid, pl.ds(row0, RPT)])

    @pl.when(sid < NCH)
    def _():
      pltpu.sync_copy(rs_sh.at[pl.ds(sid * 8, 8)],
                      rsum_hbm.at[cid, pl.ds(sid * 8, 8)])

  return k


def _tc1(NP, F_IN, HD, NA, R):
  def body(x_ref, w_ref, b_ref, abd_ref, h_ref, eler_ref):
    h = jnp.dot(x_ref[...], w_ref[...],
                preferred_element_type=jnp.float32,
                precision=lax.Precision.HIGHEST) + b_ref[...]
    h_ref[...] = h
    eler_ref[...] = jnp.dot(h, abd_ref[...],
                            preferred_element_type=jnp.float32,
                            precision=lax.Precision.HIGHEST)

  return pl.pallas_call(
      body,
      grid=(NP // R,),
      in_specs=[
          pl.BlockSpec((R, F_IN), lambda i: (i, 0)),
          pl.BlockSpec((F_IN, HD), lambda i: (0, 0)),
          pl.BlockSpec((1, HD), lambda i: (0, 0)),
          pl.BlockSpec((HD, NA), lambda i: (0, 0)),
      ],
      out_specs=[
          pl.BlockSpec((R, HD), lambda i: (i, 0)),
          pl.BlockSpec((R, NA), lambda i: (i, 0)),
      ],
      out_shape=[
          jax.ShapeDtypeStruct((NP, HD), jnp.float32),
          jax.ShapeDtypeStruct((NP, NA), jnp.float32),
      ],
  )


def _tc2(NP, NH, NHID, NCLASS, R):
  NPAIR = NH // 2

  def body(hs_ref, rs_ref, w2_ref, b2_ref, a2_ref, g_ref, eler_ref):
    acc = jnp.zeros((R, NCLASS), jnp.float32)
    for p in range(NPAIR):
      sp = hs_ref[0, p] + hs_ref[1, p]
      for u in range(2):
        h = 2 * p + u
        sh = sp[:, u * NHID:(u + 1) * NHID]
        rh = rs_ref[0, h] + rs_ref[1, h]
        x2 = sh / rh
        x2 = jnp.where(x2 > 0, x2, jnp.exp(x2) - 1.0)
        acc = acc + jnp.dot(x2, w2_ref[h],
                            preferred_element_type=jnp.float32,
                            precision=lax.Precision.HIGHEST)
    g = acc + b2_ref[...]
    g_ref[...] = jnp.concatenate(
        [g, jnp.zeros((R, FW - NCLASS), jnp.float32)], axis=1)
    eler_ref[...] = jnp.dot(g, a2_ref[...],
                            preferred_element_type=jnp.float32,
                            precision=lax.Precision.HIGHEST)

  return pl.pallas_call(
      body,
      grid=(NP // R,),
      in_specs=[
          pl.BlockSpec((2, NPAIR, R, 2 * NHID), lambda i: (0, 0, i, 0)),
          pl.BlockSpec((2, NH, R, 1), lambda i: (0, 0, i, 0)),
          pl.BlockSpec((NH, NHID, NCLASS), lambda i: (0, 0, 0)),
          pl.BlockSpec((1, NCLASS), lambda i: (0, 0)),
          pl.BlockSpec((NCLASS, 2), lambda i: (0, 0)),
      ],
      out_specs=[
          pl.BlockSpec((R, FW), lambda i: (i, 0)),
          pl.BlockSpec((R, 2), lambda i: (i, 0)),
      ],
      out_shape=[
          jax.ShapeDtypeStruct((NP, FW), jnp.float32),
          jax.ShapeDtypeStruct((NP, 2), jnp.float32),
      ],
  )


def _tc3(NP, NCLASS, R):
  def body(os_ref, rs_ref, out_ref):
    o = (os_ref[0] + os_ref[1]) / (rs_ref[0] + rs_ref[1])
    m = jnp.max(o, axis=1, keepdims=True)
    z = o - m
    lse = jnp.log(jnp.sum(jnp.exp(z), axis=1, keepdims=True))
    out_ref[...] = z - lse

  return pl.pallas_call(
      body,
      grid=(NP // R,),
      in_specs=[
          pl.BlockSpec((2, R, NCLASS), lambda i: (0, i, 0)),
          pl.BlockSpec((2, R, 1), lambda i: (0, i, 0)),
      ],
      out_specs=pl.BlockSpec((R, NCLASS), lambda i: (i, 0)),
      out_shape=jax.ShapeDtypeStruct((NP, NCLASS), jnp.float32),
  )


def kernel(features, edge_list, W1, b1, a1, W2, b2, a2):
  N, F_IN = features.shape
  E = edge_list.shape[1]
  NH, NHID, _ = W1.shape
  NCLASS = W2.shape[0]
  HD = NH * NHID
  NPAIR = NH // 2

  NP = ((N + NSUB * BLK - 1) // (NSUB * BLK)) * (NSUB * BLK)
  EPT = ((E + NW * BLK * CHB - 1) // (NW * BLK * CHB)) * BLK * CHB
  nblk = EPT // BLK
  EPAD = EPT * NW

  src = edge_list[0]
  dst = edge_list[1]
  pad = jnp.zeros((EPAD - E,), jnp.int32)
  srcp = jnp.concatenate([src, pad]).reshape(NW, nblk, BLK)
  dstp = jnp.concatenate([dst, pad]).reshape(NW, nblk, BLK)

  xp = jnp.concatenate(
      [features, jnp.zeros((NP - N, F_IN), jnp.float32)], axis=0)

  W1T = W1.reshape(HD, F_IN).T
  b1f = b1.reshape(1, HD)
  al = a1[:, :NHID]
  ar = a1[:, NHID:]
  eye = jnp.eye(NH, dtype=jnp.float32)
  abd_l = (al[:, :, None] * eye[:, None, :]).reshape(HD, NH)
  abd_r = (ar[:, :, None] * eye[:, None, :]).reshape(HD, NH)
  abd = jnp.concatenate([abd_l, abd_r], axis=1)

  h_all, eler = _tc1(NP, F_IN, HD, 2 * NH, 512)(xp, W1T, b1f, abd)
  elT = eler[:, :NH].T
  erT = eler[:, NH:].T
  hpack = h_all.reshape(NP, NPAIR, FW).transpose(1, 0, 2).reshape(
      NPAIR * NP, FW)

  evals, rsum = _sc_attn(NPAIR, 2, NP, E, EPT, nblk)(srcp, dstp, elT, erT)
  rsum = rsum.reshape(NSC, NH, NP)

  hsum = _sc_scatter(NPAIR, 2, NP, EPT, nblk)(srcp, dstp, evals, hpack)

  W2r = W2.reshape(NCLASS, NH, NHID).transpose(1, 2, 0)
  b2f = b2.reshape(1, NCLASS)
  a2lr = jnp.stack([a2[:NCLASS], a2[NCLASS:]], axis=1)
  g, eler2 = _tc2(NP, NH, NHID, NCLASS, 256)(
      hsum, rsum[..., None], W2r, b2f, a2lr)
  el2T = eler2[:, 0].reshape(1, NP)
  er2T = eler2[:, 1].reshape(1, NP)

  osum, rsum2 = _sc_edge_stage2(NP, E, EPT, nblk, NCLASS)(
      srcp, dstp, el2T, er2T, g)
  osum = osum.reshape(NSC, NP, NCLASS)
  rsum2 = rsum2.reshape(NSC, NP)

  out = _tc3(NP, NCLASS, 512)(osum, rsum2[..., None])
  return out[:N]


# speedup vs baseline: 1.0558x; 1.0558x over previous
import functools

import jax
import jax.numpy as jnp
from jax import lax
from jax.experimental import pallas as pl
from jax.experimental.pallas import tpu as pltpu
from jax.experimental.pallas import tpu_sc as plsc

ALPHA = 0.2
LANES = 16
NSUB = 16
NSC = 2
NW = NSC * NSUB
BLK = 128
FW = 128
CHB = 16


def _sc_attn(P, SUB, NP, E, EPT, nblk):
  mesh = plsc.VectorSubcoreMesh(core_axis_name="c", subcore_axis_name="s")
  NPB = NP // BLK
  NRS = SUB * NPB
  NCH = NRS // 8

  @functools.partial(
      pl.kernel,
      out_type=(
          jax.ShapeDtypeStruct((NW, P, nblk, SUB, BLK), jnp.float32),
          jax.ShapeDtypeStruct((NSC, P, NRS, BLK), jnp.float32),
      ),
      mesh=mesh,
      compiler_params=pltpu.CompilerParams(needs_layout_passes=False),
      scratch_types=[
          pltpu.VMEM((nblk, BLK), jnp.int32),
          pltpu.VMEM((nblk, BLK), jnp.int32),
          pltpu.VMEM((SUB * NP,), jnp.float32),
          pltpu.VMEM((SUB * NP,), jnp.float32),
          pltpu.VMEM((NRS, BLK), jnp.float32),
          pltpu.VMEM((nblk, SUB, BLK), jnp.float32),
          pltpu.VMEM((SUB, NPB), jnp.int32),
          pltpu.VMEM((8, BLK), jnp.float32),
          pltpu.VMEM_SHARED((NRS, BLK), jnp.float32),
      ],
  )
  def k(src_hbm, dst_hbm, elT_hbm, erT_hbm, e_hbm, rsum_hbm,
        src_c, dst_c, el_c, er_c, rs, e_all, rsidx, zrs, rs_sh):
    cid = lax.axis_index("c")
    sid = lax.axis_index("s")
    wid = cid * NSUB + sid
    base = wid * EPT

    pltpu.sync_copy(src_hbm.at[wid], src_c)
    pltpu.sync_copy(dst_hbm.at[wid], dst_c)

    @pl.loop(0, 8)
    def _(r):
      for q in range(BLK // LANES):
        zrs[r, pl.ds(LANES * q, LANES)] = jnp.zeros((LANES,), jnp.float32)

    @pl.loop(0, NPB // LANES)
    def _(i):
      for u in range(SUB):
        rsidx[u, pl.ds(LANES * i, LANES)] = (
            u * NPB + LANES * i + lax.iota(jnp.int32, 16))

    @pl.loop(0, P)
    def _(p):
      plsc.subcore_barrier()
      for z in range((NCH + NSUB - 1) // NSUB):
        ch = sid + z * NSUB
        @pl.when(ch < NCH)
        def _():
          pltpu.sync_copy(zrs, rs_sh.at[pl.ds(ch * 8, 8)])
      plsc.subcore_barrier()

      for u in range(SUB):
        pltpu.sync_copy(elT_hbm.at[p * SUB + u], el_c.at[pl.ds(u * NP, NP)])
        pltpu.sync_copy(erT_hbm.at[p * SUB + u], er_c.at[pl.ds(u * NP, NP)])

      @pl.loop(0, NRS)
      def _(r):
        for q in range(BLK // LANES):
          rs[r, pl.ds(LANES * q, LANES)] = jnp.zeros((LANES,), jnp.float32)

      @pl.loop(0, nblk)
      def _(b):
        for i in range(BLK // LANES):
          sl = pl.ds(LANES * i, LANES)
          s16 = src_c[b, sl]
          d16 = dst_c[b, sl]
          gid = base + b * BLK + LANES * i + lax.iota(jnp.int32, 16)
          valid = gid < E
          srow = lax.shift_right_logical(s16, 7)
          slane = jnp.bitwise_and(s16, 127)
          for u in range(SUB):
            el16 = plsc.load_gather(el_c, [s16 + u * NP])
            er16 = plsc.load_gather(er_c, [d16 + u * NP])
            t = el16 + er16
            e16 = jnp.exp(jnp.where(t > 0, t, ALPHA * t))
            e16 = jnp.where(valid, e16, 0.0)
            plsc.addupdate_scatter(rs, [u * NPB + srow, slane], e16)
            e_all[b, u, sl] = e16

      pltpu.sync_copy(e_all, e_hbm.at[wid, p])

      for u in range(SUB):
        pltpu.sync_copy(rs.at[pl.ds(u * NPB, NPB)],
                        rs_sh.at[rsidx.at[u]], add=True)
      plsc.subcore_barrier()

      for z in range((NCH + NSUB - 1) // NSUB):
        ch = sid + z * NSUB
        @pl.when(ch < NCH)
        def _():
          pltpu.sync_copy(rs_sh.at[pl.ds(ch * 8, 8)],
                          rsum_hbm.at[cid, p, pl.ds(ch * 8, 8)])

  return k


def _sc_scatter(P, SUB, NP, EPT, nblk):
  mesh = plsc.VectorSubcoreMesh(core_axis_name="c", subcore_axis_name="s")
  SEGW = FW // SUB
  RPT = NP // NSUB
  NCHK = nblk // CHB
  K = CHB // 2

  @functools.partial(
      pl.kernel,
      out_type=jax.ShapeDtypeStruct((NSC, P, NP, FW), jnp.float32),
      mesh=mesh,
      compiler_params=pltpu.CompilerParams(needs_layout_passes=False),
      scratch_types=[
          pltpu.VMEM((CHB, BLK), jnp.int32),
          pltpu.VMEM((CHB, BLK), jnp.int32),
          pltpu.VMEM((CHB, SUB, BLK), jnp.float32),
          pltpu.VMEM((BLK, FW), jnp.float32),
          pltpu.VMEM((BLK, FW), jnp.float32),
          pltpu.VMEM((1, BLK), jnp.int32),
          pltpu.VMEM((1, BLK), jnp.int32),
          pltpu.VMEM((1, BLK), jnp.int32),
          pltpu.VMEM((1, BLK), jnp.int32),
          pltpu.VMEM((32, FW), jnp.float32),
          pltpu.SemaphoreType.DMA,
          pltpu.SemaphoreType.DMA,
          pltpu.SemaphoreType.DMA,
          pltpu.SemaphoreType.DMA,
          pltpu.SemaphoreType.DMA,
          pltpu.VMEM_SHARED((NP, FW), jnp.float32),
      ],
  )
  def k(src_hbm, dst_hbm, e_hbm, hpack_hbm, hsum_hbm,
        src_cc, dst_cc, e_c, hrows_a, hrows_b, gidx_a, gidx_b, sidx_a,
        sidx_b, zbuf, sem_ga, sem_gb, sem_sa, sem_sb, sem_z, acc_sh):
    cid = lax.axis_index("c")
    sid = lax.axis_index("s")
    wid = cid * NSUB + sid
    row0 = sid * RPT
    dummy = hpack_hbm.at[pl.ds(0, BLK)]

    @pl.loop(0, 32)
    def _(r):
      for q in range(FW // LANES):
        zbuf[r, pl.ds(LANES * q, LANES)] = jnp.zeros((LANES,), jnp.float32)

    def mkidx(bb, p, gx, sx):
      for i in range(BLK // LANES):
        sl = pl.ds(LANES * i, LANES)
        gx[0, sl] = dst_cc[bb, sl] + p * NP
        sx[0, sl] = src_cc[bb, sl]

    def scale(hr, bb):
      @pl.loop(0, BLK // LANES)
      def _(j):
        e16s = [e_c[bb, u, pl.ds(LANES * j, LANES)] for u in range(SUB)]
        for rr in range(LANES):
          r = LANES * j + rr
          for u in range(SUB):
            ev = e16s[u][rr]
            for q in range(SEGW // LANES):
              qs = pl.ds(u * SEGW + LANES * q, LANES)
              hr[r, qs] = hr[r, qs] * ev

    @pl.loop(0, P)
    def _(p):
      plsc.subcore_barrier()
      zs = [pltpu.async_copy(zbuf, acc_sh.at[pl.ds(row0 + z * 32, 32)],
                             sem_z) for z in range(RPT // 32)]
      for z in zs:
        z.wait()
      plsc.subcore_barrier()

      @pl.loop(0, NCHK)
      def _(c):
        pltpu.sync_copy(src_hbm.at[wid, pl.ds(c * CHB, CHB)], src_cc)
        pltpu.sync_copy(dst_hbm.at[wid, pl.ds(c * CHB, CHB)], dst_cc)
        pltpu.sync_copy(e_hbm.at[wid, p, pl.ds(c * CHB, CHB)], e_c)

        mkidx(0, p, gidx_a, sidx_a)
        pltpu.async_copy(hpack_hbm.at[gidx_a.at[0]], hrows_a, sem_ga)

        @pl.loop(0, K)
        def _(kk):
          @pl.when(kk > 0)
          def _():
            pltpu.make_async_copy(dummy, hrows_b, sem_sb).wait()
          mkidx(2 * kk + 1, p, gidx_b, sidx_b)
          gb = pltpu.async_copy(hpack_hbm.at[gidx_b.at[0]], hrows_b, sem_gb)

          pltpu.make_async_copy(dummy, hrows_a, sem_ga).wait()
          scale(hrows_a, 2 * kk)
          sa = pltpu.async_copy(hrows_a, acc_sh.at[sidx_a.at[0]], sem_sa,
                                add=True)

          gb.wait()
          scale(hrows_b, 2 * kk + 1)
          pltpu.async_copy(hrows_b, acc_sh.at[sidx_b.at[0]], sem_sb,
                           add=True)

          sa.wait()
          @pl.when(kk < K - 1)
          def _():
            mkidx(2 * kk + 2, p, gidx_a, sidx_a)
            pltpu.async_copy(hpack_hbm.at[gidx_a.at[0]], hrows_a, sem_ga)

        pltpu.make_async_copy(dummy, hrows_b, sem_sb).wait()

      plsc.subcore_barrier()
      pltpu.sync_copy(acc_sh.at[pl.ds(row0, RPT)],
                      hsum_hbm.at[cid, p, pl.ds(row0, RPT)])

  return k


def _sc_edge_stage2(NP, E, EPT, nblk, NCLASS):
  mesh = plsc.VectorSubcoreMesh(core_axis_name="c", subcore_axis_name="s")
  NPK = NP // 8
  RPT = NPK // NSUB
  NPB = NP // BLK
  NCH = NPB // 8

  @functools.partial(
      pl.kernel,
      out_type=(
          jax.ShapeDtypeStruct((NSC, NPK, FW), jnp.float32),
          jax.ShapeDtypeStruct((NSC, NPB, BLK), jnp.float32),
      ),
      mesh=mesh,
      compiler_params=pltpu.CompilerParams(needs_layout_passes=False),
      scratch_types=[
          pltpu.VMEM((nblk, BLK), jnp.int32),
          pltpu.VMEM((nblk, BLK), jnp.int32),
          pltpu.VMEM((NP,), jnp.float32),
          pltpu.VMEM((NP,), jnp.float32),
          pltpu.VMEM((NPB, BLK), jnp.float32),
          pltpu.VMEM((BLK, FW), jnp.float32),
          pltpu.VMEM((BLK, FW), jnp.float32),
          pltpu.VMEM((BLK, FW), jnp.float32),
          pltpu.VMEM((BLK, FW), jnp.float32),
          pltpu.VMEM((2, BLK), jnp.int32),
          pltpu.VMEM((2, BLK), jnp.int32),
          pltpu.VMEM((2, BLK), jnp.int32),
          pltpu.VMEM((1, NPB), jnp.int32),
          pltpu.VMEM((2, BLK), jnp.float32),
          pltpu.VMEM((8, FW), jnp.float32),
          pltpu.SemaphoreType.DMA,
          pltpu.SemaphoreType.DMA,
          pltpu.SemaphoreType.DMA,
          pltpu.SemaphoreType.DMA,
          pltpu.VMEM_SHARED((NPK, FW), jnp.float32),
          pltpu.VMEM_SHARED((NPB, BLK), jnp.float32),
      ],
  )
  def k(src_hbm, dst_hbm, el_hbm, er_hbm, g_hbm, osum_hbm, rsum_hbm,
        src_c, dst_c, el_c, er_c, rs, hrows_a, hrows_b, srows_a, srows_b,
        gidx, sidx, soff, rsidx, e_blk, zbuf, sem_ga, sem_gb, sem_sa,
        sem_sb, acc_sh, rs_sh):
    cid = lax.axis_index("c")
    sid = lax.axis_index("s")
    wid = cid * NSUB + sid
    base = wid * EPT
    row0 = sid * RPT
    dummy = g_hbm.at[pl.ds(0, BLK)]
    K = nblk // 2

    pltpu.sync_copy(src_hbm.at[wid], src_c)
    pltpu.sync_copy(dst_hbm.at[wid], dst_c)
    pltpu.sync_copy(el_hbm.at[0], el_c)
    pltpu.sync_copy(er_hbm.at[0], er_c)

    @pl.loop(0, 8)
    def _(r):
      for q in range(FW // LANES):
        zbuf[r, pl.ds(LANES * q, LANES)] = jnp.zeros((LANES,), jnp.float32)

    for srows in (srows_a, srows_b):
      @pl.loop(0, BLK)
      def _(r, srows=srows):
        for q in range(FW // LANES):
          srows[r, pl.ds(LANES * q, LANES)] = jnp.zeros((LANES,),
                                                        jnp.float32)

    @pl.loop(0, NPB // LANES)
    def _(i):
      rsidx[0, pl.ds(LANES * i, LANES)] = (
          LANES * i + lax.iota(jnp.int32, 16))

    @pl.loop(0, NPB)
    def _(r):
      for q in range(BLK // LANES):
        rs[r, pl.ds(LANES * q, LANES)] = jnp.zeros((LANES,), jnp.float32)

    for z in range(RPT // 8):
      pltpu.sync_copy(zbuf, acc_sh.at[pl.ds(row0 + z * 8, 8)])

    @pl.when(sid < NCH)
    def _():
      pltpu.sync_copy(zbuf.at[pl.ds(0, 8)], rs_sh.at[pl.ds(sid * 8, 8)])
    plsc.subcore_barrier()

    def prep(b, v):
      for i in range(BLK // LANES):
        sl = pl.ds(LANES * i, LANES)
        s16 = src_c[b, sl]
        d16 = dst_c[b, sl]
        gid = base + b * BLK + LANES * i + lax.iota(jnp.int32, 16)
        el16 = plsc.load_gather(el_c, [s16])
        er16 = plsc.load_gather(er_c, [d16])
        t = el16 + er16
        e16 = jnp.exp(jnp.where(t > 0, t, ALPHA * t))
        e16 = jnp.where(gid < E, e16, 0.0)
        plsc.addupdate_scatter(
            rs, [lax.shift_right_logical(s16, 7), jnp.bitwise_and(s16, 127)],
            e16)
        e_blk[v, sl] = e16
        gidx[v, sl] = d16
        sidx[v, sl] = lax.shift_right_logical(s16, 3)
        soff[v, sl] = jnp.bitwise_and(s16, 7) * LANES

    def scale(hrows, srows, v):
      @pl.loop(0, BLK // LANES)
      def _(j):
        e16 = e_blk[v, pl.ds(LANES * j, LANES)]
        o16 = soff[v, pl.ds(LANES * j, LANES)]
        for rr in range(LANES):
          r = LANES * j + rr
          w = hrows[r, pl.ds(0, LANES)] * e16[rr]
          srows[r, pl.ds(o16[rr], LANES)] = w

    def unscale(srows, v):
      @pl.loop(0, BLK // LANES)
      def _(j):
        o16 = soff[v, pl.ds(LANES * j, LANES)]
        for rr in range(LANES):
          srows[LANES * j + rr, pl.ds(o16[rr], LANES)] = jnp.zeros(
              (LANES,), jnp.float32)

    prep(0, 0)
    pltpu.async_copy(g_hbm.at[gidx.at[0]], hrows_a, sem_ga)

    @pl.loop(0, K)
    def _(kk):
      @pl.when(kk > 0)
      def _():
        pltpu.make_async_copy(dummy, srows_b, sem_sb).wait()
        unscale(srows_b, 1)
      prep(2 * kk + 1, 1)
      gb = pltpu.async_copy(g_hbm.at[gidx.at[1]], hrows_b, sem_gb)

      pltpu.make_async_copy(dummy, hrows_a, sem_ga).wait()
      scale(hrows_a, srows_a, 0)
      sa = pltpu.async_copy(srows_a, acc_sh.at[sidx.at[0]], sem_sa,
                            add=True)

      gb.wait()
      scale(hrows_b, srows_b, 1)
      pltpu.async_copy(srows_b, acc_sh.at[sidx.at[1]], sem_sb, add=True)

      sa.wait()
      unscale(srows_a, 0)
      @pl.when(kk < K - 1)
      def _():
        prep(2 * kk + 2, 0)
        pltpu.async_copy(g_hbm.at[gidx.at[0]], hrows_a, sem_ga)

    pltpu.make_async_copy(dummy, srows_b, sem_sb).wait()

    pltpu.sync_copy(rs, rs_sh.at[rsidx.at[0]], add=True)
    plsc.subcore_barrier()

    pltpu.sync_copy(acc_sh.at[pl.ds(row0, RPT)],
                    osum_hbm.at[cid, pl.ds(row0, RPT)])

    @pl.when(sid < NCH)
    def _():
      pltpu.sync_copy(rs_sh.at[pl.ds(sid * 8, 8)],
                      rsum_hbm.at[cid, pl.ds(sid * 8, 8)])

  return k


def _tc1(NP, F_IN, HD, NA, R):
  def body(x_ref, w_ref, b_ref, abd_ref, h_ref, eler_ref):
    h = jnp.dot(x_ref[...], w_ref[...],
                preferred_element_type=jnp.float32,
                precision=lax.Precision.HIGHEST) + b_ref[...]
    h_ref[...] = h
    eler_ref[...] = jnp.dot(h, abd_ref[...],
                            preferred_element_type=jnp.float32,
                            precision=lax.Precision.HIGHEST)

  return pl.pallas_call(
      body,
      grid=(NP // R,),
      in_specs=[
          pl.BlockSpec((R, F_IN), lambda i: (i, 0)),
          pl.BlockSpec((F_IN, HD), lambda i: (0, 0)),
          pl.BlockSpec((1, HD), lambda i: (0, 0)),
          pl.BlockSpec((HD, NA), lambda i: (0, 0)),
      ],
      out_specs=[
          pl.BlockSpec((R, HD), lambda i: (i, 0)),
          pl.BlockSpec((R, NA), lambda i: (i, 0)),
      ],
      out_shape=[
          jax.ShapeDtypeStruct((NP, HD), jnp.float32),
          jax.ShapeDtypeStruct((NP, NA), jnp.float32),
      ],
  )


def _tc2(NP, NH, NHID, NCLASS, R):
  NPAIR = NH // 2

  def body(hs_ref, rs_ref, w2_ref, b2_ref, a2_ref, g_ref, eler_ref):
    acc = jnp.zeros((R, NCLASS), jnp.float32)
    for p in range(NPAIR):
      sp = hs_ref[0, p] + hs_ref[1, p]
      for u in range(2):
        h = 2 * p + u
        sh = sp[:, u * NHID:(u + 1) * NHID]
        rh = rs_ref[0, h] + rs_ref[1, h]
        x2 = sh / rh
        x2 = jnp.where(x2 > 0, x2, jnp.exp(x2) - 1.0)
        acc = acc + jnp.dot(x2, w2_ref[h],
                            preferred_element_type=jnp.float32,
                            precision=lax.Precision.HIGHEST)
    g = acc + b2_ref[...]
    g_ref[...] = jnp.concatenate(
        [g, jnp.zeros((R, FW - NCLASS), jnp.float32)], axis=1)
    eler_ref[...] = jnp.dot(g, a2_ref[...],
                            preferred_element_type=jnp.float32,
                            precision=lax.Precision.HIGHEST)

  return pl.pallas_call(
      body,
      grid=(NP // R,),
      in_specs=[
          pl.BlockSpec((2, NPAIR, R, 2 * NHID), lambda i: (0, 0, i, 0)),
          pl.BlockSpec((2, NH, R, 1), lambda i: (0, 0, i, 0)),
          pl.BlockSpec((NH, NHID, NCLASS), lambda i: (0, 0, 0)),
          pl.BlockSpec((1, NCLASS), lambda i: (0, 0)),
          pl.BlockSpec((NCLASS, 2), lambda i: (0, 0)),
      ],
      out_specs=[
          pl.BlockSpec((R, FW), lambda i: (i, 0)),
          pl.BlockSpec((R, 2), lambda i: (i, 0)),
      ],
      out_shape=[
          jax.ShapeDtypeStruct((NP, FW), jnp.float32),
          jax.ShapeDtypeStruct((NP, 2), jnp.float32),
      ],
  )


def _tc3(NP, NCLASS, R):
  def body(os_ref, rs_ref, out_ref):
    o = (os_ref[0] + os_ref[1]) / (rs_ref[0] + rs_ref[1])
    m = jnp.max(o, axis=1, keepdims=True)
    z = o - m
    lse = jnp.log(jnp.sum(jnp.exp(z), axis=1, keepdims=True))
    out_ref[...] = z - lse

  return pl.pallas_call(
      body,
      grid=(NP // R,),
      in_specs=[
          pl.BlockSpec((2, R, NCLASS), lambda i: (0, i, 0)),
          pl.BlockSpec((2, R, 1), lambda i: (0, i, 0)),
      ],
      out_specs=pl.BlockSpec((R, NCLASS), lambda i: (i, 0)),
      out_shape=jax.ShapeDtypeStruct((NP, NCLASS), jnp.float32),
  )


def kernel(features, edge_list, W1, b1, a1, W2, b2, a2):
  N, F_IN = features.shape
  E = edge_list.shape[1]
  NH, NHID, _ = W1.shape
  NCLASS = W2.shape[0]
  HD = NH * NHID
  NPAIR = NH // 2

  NP = ((N + NSUB * BLK - 1) // (NSUB * BLK)) * (NSUB * BLK)
  EPT = ((E + NW * BLK * CHB - 1) // (NW * BLK * CHB)) * BLK * CHB
  nblk = EPT // BLK
  EPAD = EPT * NW

  src = edge_list[0]
  dst = edge_list[1]
  pad = jnp.zeros((EPAD - E,), jnp.int32)
  srcp = jnp.concatenate([src, pad]).reshape(NW, nblk, BLK)
  dstp = jnp.concatenate([dst, pad]).reshape(NW, nblk, BLK)

  xp = jnp.concatenate(
      [features, jnp.zeros((NP - N, F_IN), jnp.float32)], axis=0)

  W1T = W1.reshape(HD, F_IN).T
  b1f = b1.reshape(1, HD)
  al = a1[:, :NHID]
  ar = a1[:, NHID:]
  eye = jnp.eye(NH, dtype=jnp.float32)
  abd_l = (al[:, :, None] * eye[:, None, :]).reshape(HD, NH)
  abd_r = (ar[:, :, None] * eye[:, None, :]).reshape(HD, NH)
  abd = jnp.concatenate([abd_l, abd_r], axis=1)

  h_all, eler = _tc1(NP, F_IN, HD, 2 * NH, 512)(xp, W1T, b1f, abd)
  elT = eler[:, :NH].T
  erT = eler[:, NH:].T
  hpack = h_all.reshape(NP, NPAIR, FW).transpose(1, 0, 2).reshape(
      NPAIR * NP, FW)

  evals, rsum = _sc_attn(NPAIR, 2, NP, E, EPT, nblk)(srcp, dstp, elT, erT)
  rsum = rsum.reshape(NSC, NH, NP)

  hsum = _sc_scatter(NPAIR, 2, NP, EPT, nblk)(srcp, dstp, evals, hpack)

  W2r = W2.reshape(NCLASS, NH, NHID).transpose(1, 2, 0)
  b2f = b2.reshape(1, NCLASS)
  a2lr = jnp.stack([a2[:NCLASS], a2[NCLASS:]], axis=1)
  g, eler2 = _tc2(NP, NH, NHID, NCLASS, 256)(
      hsum, rsum[..., None], W2r, b2f, a2lr)
  el2T = eler2[:, 0].reshape(1, NP)
  er2T = eler2[:, 1].reshape(1, NP)

  osum, rsum2 = _sc_edge_stage2(NP, E, EPT, nblk, NCLASS)(
      srcp, dstp, el2T, er2T, g)
  osum = osum.reshape(NSC, NP, NCLASS)
  rsum2 = rsum2.reshape(NSC, NP)

  out = _tc3(NP, NCLASS, 512)(osum, rsum2[..., None])
  return out[:N]


# speedup vs baseline: 2.5586x; 2.4234x over previous
import functools

import jax
import jax.numpy as jnp
from jax import lax
from jax.experimental import pallas as pl
from jax.experimental.pallas import tpu as pltpu
from jax.experimental.pallas import tpu_sc as plsc

ALPHA = 0.2
LANES = 16
NSUB = 16
NSC = 2
NW = NSC * NSUB
BLK = 128
FW = 128
CHB = 16


def _sc_attn(P, SUB, NP, E, EPT, nblk):
  mesh = plsc.VectorSubcoreMesh(core_axis_name="c", subcore_axis_name="s")
  NPB = NP // BLK
  NRS = SUB * NPB
  NCH = NRS // 8

  @functools.partial(
      pl.kernel,
      out_type=(
          jax.ShapeDtypeStruct((NW, P, nblk, SUB, BLK), jnp.float32),
          jax.ShapeDtypeStruct((NSC, P, NRS, BLK), jnp.float32),
      ),
      mesh=mesh,
      compiler_params=pltpu.CompilerParams(needs_layout_passes=False),
      scratch_types=[
          pltpu.VMEM((nblk, BLK), jnp.int32),
          pltpu.VMEM((nblk, BLK), jnp.int32),
          pltpu.VMEM((SUB * NP,), jnp.float32),
          pltpu.VMEM((SUB * NP,), jnp.float32),
          pltpu.VMEM((NRS, BLK), jnp.float32),
          pltpu.VMEM((nblk, SUB, BLK), jnp.float32),
          pltpu.VMEM((SUB, NPB), jnp.int32),
          pltpu.VMEM((8, BLK), jnp.float32),
          pltpu.VMEM_SHARED((NRS, BLK), jnp.float32),
      ],
  )
  def k(src_hbm, dst_hbm, elT_hbm, erT_hbm, e_hbm, rsum_hbm,
        src_c, dst_c, el_c, er_c, rs, e_all, rsidx, zrs, rs_sh):
    cid = lax.axis_index("c")
    sid = lax.axis_index("s")
    wid = cid * NSUB + sid
    base = wid * EPT

    pltpu.sync_copy(src_hbm.at[wid], src_c)
    pltpu.sync_copy(dst_hbm.at[wid], dst_c)

    @pl.loop(0, 8)
    def _(r):
      for q in range(BLK // LANES):
        zrs[r, pl.ds(LANES * q, LANES)] = jnp.zeros((LANES,), jnp.float32)

    @pl.loop(0, NPB // LANES)
    def _(i):
      for u in range(SUB):
        rsidx[u, pl.ds(LANES * i, LANES)] = (
            u * NPB + LANES * i + lax.iota(jnp.int32, 16))

    @pl.loop(0, P)
    def _(p):
      plsc.subcore_barrier()
      for z in range((NCH + NSUB - 1) // NSUB):
        ch = sid + z * NSUB
        @pl.when(ch < NCH)
        def _():
          pltpu.sync_copy(zrs, rs_sh.at[pl.ds(ch * 8, 8)])
      plsc.subcore_barrier()

      for u in range(SUB):
        pltpu.sync_copy(elT_hbm.at[p * SUB + u], el_c.at[pl.ds(u * NP, NP)])
        pltpu.sync_copy(erT_hbm.at[p * SUB + u], er_c.at[pl.ds(u * NP, NP)])

      @pl.loop(0, NRS)
      def _(r):
        for q in range(BLK // LANES):
          rs[r, pl.ds(LANES * q, LANES)] = jnp.zeros((LANES,), jnp.float32)

      @pl.loop(0, nblk)
      def _(b):
        for i in range(BLK // LANES):
          sl = pl.ds(LANES * i, LANES)
          s16 = src_c[b, sl]
          d16 = dst_c[b, sl]
          gid = base + b * BLK + LANES * i + lax.iota(jnp.int32, 16)
          valid = gid < E
          srow = lax.shift_right_logical(s16, 7)
          slane = jnp.bitwise_and(s16, 127)
          for u in range(SUB):
            el16 = plsc.load_gather(el_c, [s16 + u * NP])
            er16 = plsc.load_gather(er_c, [d16 + u * NP])
            t = el16 + er16
            e16 = jnp.exp(jnp.where(t > 0, t, ALPHA * t))
            e16 = jnp.where(valid, e16, 0.0)
            plsc.addupdate_scatter(rs, [u * NPB + srow, slane], e16)
            e_all[b, u, sl] = e16

      pltpu.sync_copy(e_all, e_hbm.at[wid, p])

      for u in range(SUB):
        pltpu.sync_copy(rs.at[pl.ds(u * NPB, NPB)],
                        rs_sh.at[rsidx.at[u]], add=True)
      plsc.subcore_barrier()

      for z in range((NCH + NSUB - 1) // NSUB):
        ch = sid + z * NSUB
        @pl.when(ch < NCH)
        def _():
          pltpu.sync_copy(rs_sh.at[pl.ds(ch * 8, 8)],
                          rsum_hbm.at[cid, p, pl.ds(ch * 8, 8)])

  return k


def _sc_scatter(P, SUB, NP, EPT, nblk):
  mesh = plsc.VectorSubcoreMesh(core_axis_name="c", subcore_axis_name="s")
  SEGW = FW // SUB
  RPT = NP // NSUB
  NCHK = nblk // CHB
  K = CHB // 2

  @functools.partial(
      pl.kernel,
      out_type=jax.ShapeDtypeStruct((NSC, P, NP, FW), jnp.float32),
      mesh=mesh,
      compiler_params=pltpu.CompilerParams(needs_layout_passes=False),
      scratch_types=[
          pltpu.VMEM((CHB, BLK), jnp.int32),
          pltpu.VMEM((CHB, BLK), jnp.int32),
          pltpu.VMEM((CHB, SUB, BLK), jnp.float32),
          pltpu.VMEM((BLK, FW), jnp.float32),
          pltpu.VMEM((BLK, FW), jnp.float32),
          pltpu.VMEM((1, BLK), jnp.int32),
          pltpu.VMEM((1, BLK), jnp.int32),
          pltpu.VMEM((1, BLK), jnp.int32),
          pltpu.VMEM((1, BLK), jnp.int32),
          pltpu.VMEM((32, FW), jnp.float32),
          pltpu.SemaphoreType.DMA,
          pltpu.SemaphoreType.DMA,
          pltpu.SemaphoreType.DMA,
          pltpu.SemaphoreType.DMA,
          pltpu.SemaphoreType.DMA,
          pltpu.VMEM_SHARED((NP, FW), jnp.float32),
      ],
  )
  def k(src_hbm, dst_hbm, e_hbm, hpack_hbm, hsum_hbm,
        src_cc, dst_cc, e_c, hrows_a, hrows_b, gidx_a, gidx_b, sidx_a,
        sidx_b, zbuf, sem_ga, sem_gb, sem_sa, sem_sb, sem_z, acc_sh):
    cid = lax.axis_index("c")
    sid = lax.axis_index("s")
    wid = cid * NSUB + sid
    row0 = sid * RPT
    dummy = hpack_hbm.at[pl.ds(0, BLK)]

    @pl.loop(0, 32)
    def _(r):
      for q in range(FW // LANES):
        zbuf[r, pl.ds(LANES * q, LANES)] = jnp.zeros((LANES,), jnp.float32)

    def mkidx(bb, p, gx, sx):
      for i in range(BLK // LANES):
        sl = pl.ds(LANES * i, LANES)
        gx[0, sl] = dst_cc[bb, sl] + p * NP
        sx[0, sl] = src_cc[bb, sl]

    def scale(hr, bb):
      @pl.loop(0, BLK // LANES)
      def _(j):
        e16s = [e_c[bb, u, pl.ds(LANES * j, LANES)] for u in range(SUB)]
        for rr in range(LANES):
          r = LANES * j + rr
          for u in range(SUB):
            ev = e16s[u][rr]
            for q in range(SEGW // LANES):
              qs = pl.ds(u * SEGW + LANES * q, LANES)
              hr[r, qs] = hr[r, qs] * ev

    @pl.loop(0, P)
    def _(p):
      plsc.subcore_barrier()
      zs = [pltpu.async_copy(zbuf, acc_sh.at[pl.ds(row0 + z * 32, 32)],
                             sem_z) for z in range(RPT // 32)]
      for z in zs:
        z.wait()
      plsc.subcore_barrier()

      @pl.loop(0, NCHK)
      def _(c):
        pltpu.sync_copy(src_hbm.at[wid, pl.ds(c * CHB, CHB)], src_cc)
        pltpu.sync_copy(dst_hbm.at[wid, pl.ds(c * CHB, CHB)], dst_cc)
        pltpu.sync_copy(e_hbm.at[wid, p, pl.ds(c * CHB, CHB)], e_c)

        mkidx(0, p, gidx_a, sidx_a)
        pltpu.async_copy(hpack_hbm.at[gidx_a.at[0]], hrows_a, sem_ga)

        @pl.loop(0, K)
        def _(kk):
          @pl.when(kk > 0)
          def _():
            pltpu.make_async_copy(dummy, hrows_b, sem_sb).wait()
          mkidx(2 * kk + 1, p, gidx_b, sidx_b)
          gb = pltpu.async_copy(hpack_hbm.at[gidx_b.at[0]], hrows_b, sem_gb)

          pltpu.make_async_copy(dummy, hrows_a, sem_ga).wait()
          scale(hrows_a, 2 * kk)
          sa = pltpu.async_copy(hrows_a, acc_sh.at[sidx_a.at[0]], sem_sa,
                                add=True)

          gb.wait()
          scale(hrows_b, 2 * kk + 1)
          pltpu.async_copy(hrows_b, acc_sh.at[sidx_b.at[0]], sem_sb,
                           add=True)

          sa.wait()
          @pl.when(kk < K - 1)
          def _():
            mkidx(2 * kk + 2, p, gidx_a, sidx_a)
            pltpu.async_copy(hpack_hbm.at[gidx_a.at[0]], hrows_a, sem_ga)

        pltpu.make_async_copy(dummy, hrows_b, sem_sb).wait()

      plsc.subcore_barrier()
      pltpu.sync_copy(acc_sh.at[pl.ds(row0, RPT)],
                      hsum_hbm.at[cid, p, pl.ds(row0, RPT)])

  return k


def _sc_edge_stage2(NP, E, EPT, nblk, NCLASS):
  mesh = plsc.VectorSubcoreMesh(core_axis_name="c", subcore_axis_name="s")
  NPK = NP // 8
  RPT = NPK // NSUB
  NPB = NP // BLK
  NCH = NPB // 8

  @functools.partial(
      pl.kernel,
      out_type=(
          jax.ShapeDtypeStruct((NSC, NPK, FW), jnp.float32),
          jax.ShapeDtypeStruct((NSC, NPB, BLK), jnp.float32),
      ),
      mesh=mesh,
      compiler_params=pltpu.CompilerParams(needs_layout_passes=False),
      scratch_types=[
          pltpu.VMEM((nblk, BLK), jnp.int32),
          pltpu.VMEM((nblk, BLK), jnp.int32),
          pltpu.VMEM((NP,), jnp.float32),
          pltpu.VMEM((NP,), jnp.float32),
          pltpu.VMEM((NPB, BLK), jnp.float32),
          pltpu.VMEM((BLK, FW), jnp.float32),
          pltpu.VMEM((BLK, FW), jnp.float32),
          pltpu.VMEM((BLK, FW), jnp.float32),
          pltpu.VMEM((BLK, FW), jnp.float32),
          pltpu.VMEM((2, BLK), jnp.int32),
          pltpu.VMEM((2, BLK), jnp.int32),
          pltpu.VMEM((2, BLK), jnp.int32),
          pltpu.VMEM((1, NPB), jnp.int32),
          pltpu.VMEM((2, BLK), jnp.float32),
          pltpu.VMEM((8, FW), jnp.float32),
          pltpu.SemaphoreType.DMA,
          pltpu.SemaphoreType.DMA,
          pltpu.SemaphoreType.DMA,
          pltpu.SemaphoreType.DMA,
          pltpu.VMEM_SHARED((NPK, FW), jnp.float32),
          pltpu.VMEM_SHARED((NPB, BLK), jnp.float32),
      ],
  )
  def k(src_hbm, dst_hbm, el_hbm, er_hbm, g_hbm, osum_hbm, rsum_hbm,
        src_c, dst_c, el_c, er_c, rs, hrows_a, hrows_b, srows_a, srows_b,
        gidx, sidx, soff, rsidx, e_blk, zbuf, sem_ga, sem_gb, sem_sa,
        sem_sb, acc_sh, rs_sh):
    cid = lax.axis_index("c")
    sid = lax.axis_index("s")
    wid = cid * NSUB + sid
    base = wid * EPT
    row0 = sid * RPT
    dummy = g_hbm.at[pl.ds(0, BLK)]
    K = nblk // 2

    pltpu.sync_copy(src_hbm.at[wid], src_c)
    pltpu.sync_copy(dst_hbm.at[wid], dst_c)
    pltpu.sync_copy(el_hbm.at[0], el_c)
    pltpu.sync_copy(er_hbm.at[0], er_c)

    @pl.loop(0, 8)
    def _(r):
      for q in range(FW // LANES):
        zbuf[r, pl.ds(LANES * q, LANES)] = jnp.zeros((LANES,), jnp.float32)

    for srows in (srows_a, srows_b):
      @pl.loop(0, BLK)
      def _(r, srows=srows):
        for q in range(FW // LANES):
          srows[r, pl.ds(LANES * q, LANES)] = jnp.zeros((LANES,),
                                                        jnp.float32)

    @pl.loop(0, NPB // LANES)
    def _(i):
      rsidx[0, pl.ds(LANES * i, LANES)] = (
          LANES * i + lax.iota(jnp.int32, 16))

    @pl.loop(0, NPB)
    def _(r):
      for q in range(BLK // LANES):
        rs[r, pl.ds(LANES * q, LANES)] = jnp.zeros((LANES,), jnp.float32)

    for z in range(RPT // 8):
      pltpu.sync_copy(zbuf, acc_sh.at[pl.ds(row0 + z * 8, 8)])

    @pl.when(sid < NCH)
    def _():
      pltpu.sync_copy(zbuf.at[pl.ds(0, 8)], rs_sh.at[pl.ds(sid * 8, 8)])
    plsc.subcore_barrier()

    def prep(b, v):
      for i in range(BLK // LANES):
        sl = pl.ds(LANES * i, LANES)
        s16 = src_c[b, sl]
        d16 = dst_c[b, sl]
        gid = base + b * BLK + LANES * i + lax.iota(jnp.int32, 16)
        el16 = plsc.load_gather(el_c, [s16])
        er16 = plsc.load_gather(er_c, [d16])
        t = el16 + er16
        e16 = jnp.exp(jnp.where(t > 0, t, ALPHA * t))
        e16 = jnp.where(gid < E, e16, 0.0)
        plsc.addupdate_scatter(
            rs, [lax.shift_right_logical(s16, 7), jnp.bitwise_and(s16, 127)],
            e16)
        e_blk[v, sl] = e16
        gidx[v, sl] = d16
        sidx[v, sl] = lax.shift_right_logical(s16, 3)
        soff[v, sl] = jnp.bitwise_and(s16, 7) * LANES

    def scale(hrows, srows, v):
      @pl.loop(0, BLK // LANES)
      def _(j):
        e16 = e_blk[v, pl.ds(LANES * j, LANES)]
        o16 = soff[v, pl.ds(LANES * j, LANES)]
        for rr in range(LANES):
          r = LANES * j + rr
          w = hrows[r, pl.ds(0, LANES)] * e16[rr]
          srows[r, pl.ds(o16[rr], LANES)] = w

    def unscale(srows, v):
      @pl.loop(0, BLK // LANES)
      def _(j):
        o16 = soff[v, pl.ds(LANES * j, LANES)]
        for rr in range(LANES):
          srows[LANES * j + rr, pl.ds(o16[rr], LANES)] = jnp.zeros(
              (LANES,), jnp.float32)

    prep(0, 0)
    pltpu.async_copy(g_hbm.at[gidx.at[0]], hrows_a, sem_ga)

    @pl.loop(0, K)
    def _(kk):
      @pl.when(kk > 0)
      def _():
        pltpu.make_async_copy(dummy, srows_b, sem_sb).wait()
        unscale(srows_b, 1)
      prep(2 * kk + 1, 1)
      gb = pltpu.async_copy(g_hbm.at[gidx.at[1]], hrows_b, sem_gb)

      pltpu.make_async_copy(dummy, hrows_a, sem_ga).wait()
      scale(hrows_a, srows_a, 0)
      sa = pltpu.async_copy(srows_a, acc_sh.at[sidx.at[0]], sem_sa,
                            add=True)

      gb.wait()
      scale(hrows_b, srows_b, 1)
      pltpu.async_copy(srows_b, acc_sh.at[sidx.at[1]], sem_sb, add=True)

      sa.wait()
      unscale(srows_a, 0)
      @pl.when(kk < K - 1)
      def _():
        prep(2 * kk + 2, 0)
        pltpu.async_copy(g_hbm.at[gidx.at[0]], hrows_a, sem_ga)

    pltpu.make_async_copy(dummy, srows_b, sem_sb).wait()

    pltpu.sync_copy(rs, rs_sh.at[rsidx.at[0]], add=True)
    plsc.subcore_barrier()

    pltpu.sync_copy(acc_sh.at[pl.ds(row0, RPT)],
                    osum_hbm.at[cid, pl.ds(row0, RPT)])

    @pl.when(sid < NCH)
    def _():
      pltpu.sync_copy(rs_sh.at[pl.ds(sid * 8, 8)],
                      rsum_hbm.at[cid, pl.ds(sid * 8, 8)])

  return k


def _tc1(NP, F_IN, HD, NA, R):
  def body(x_ref, w_ref, b_ref, abd_ref, h_ref, eler_ref):
    h = jnp.dot(x_ref[...], w_ref[...],
                preferred_element_type=jnp.float32,
                precision=lax.Precision.HIGHEST) + b_ref[...]
    h_ref[...] = h
    eler_ref[...] = jnp.dot(h, abd_ref[...],
                            preferred_element_type=jnp.float32,
                            precision=lax.Precision.HIGHEST)

  return pl.pallas_call(
      body,
      grid=(NP // R,),
      in_specs=[
          pl.BlockSpec((R, F_IN), lambda i: (i, 0)),
          pl.BlockSpec((F_IN, HD), lambda i: (0, 0)),
          pl.BlockSpec((1, HD), lambda i: (0, 0)),
          pl.BlockSpec((HD, NA), lambda i: (0, 0)),
      ],
      out_specs=[
          pl.BlockSpec((R, HD), lambda i: (i, 0)),
          pl.BlockSpec((R, NA), lambda i: (i, 0)),
      ],
      out_shape=[
          jax.ShapeDtypeStruct((NP, HD), jnp.float32),
          jax.ShapeDtypeStruct((NP, NA), jnp.float32),
      ],
  )


def _tc2(NP, NH, NHID, NCLASS, R):
  NPAIR = NH // 2

  def body(hs_ref, rs_ref, w2_ref, b2_ref, a2_ref, g_ref, eler_ref):
    acc = jnp.zeros((R, NCLASS), jnp.float32)
    for p in range(NPAIR):
      sp = hs_ref[0, p] + hs_ref[1, p]
      for u in range(2):
        h = 2 * p + u
        sh = sp[:, u * NHID:(u + 1) * NHID]
        rh = rs_ref[0, h] + rs_ref[1, h]
        x2 = sh / rh
        x2 = jnp.where(x2 > 0, x2, jnp.exp(x2) - 1.0)
        acc = acc + jnp.dot(x2, w2_ref[h],
                            preferred_element_type=jnp.float32,
                            precision=lax.Precision.HIGHEST)
    g = acc + b2_ref[...]
    g_ref[...] = jnp.concatenate(
        [g, jnp.zeros((R, FW - NCLASS), jnp.float32)], axis=1)
    eler_ref[...] = jnp.dot(g, a2_ref[...],
                            preferred_element_type=jnp.float32,
                            precision=lax.Precision.HIGHEST)

  return pl.pallas_call(
      body,
      grid=(NP // R,),
      in_specs=[
          pl.BlockSpec((2, NPAIR, R, 2 * NHID), lambda i: (0, 0, i, 0)),
          pl.BlockSpec((2, NH, R, 1), lambda i: (0, 0, i, 0)),
          pl.BlockSpec((NH, NHID, NCLASS), lambda i: (0, 0, 0)),
          pl.BlockSpec((1, NCLASS), lambda i: (0, 0)),
          pl.BlockSpec((NCLASS, 2), lambda i: (0, 0)),
      ],
      out_specs=[
          pl.BlockSpec((R, FW), lambda i: (i, 0)),
          pl.BlockSpec((R, 2), lambda i: (i, 0)),
      ],
      out_shape=[
          jax.ShapeDtypeStruct((NP, FW), jnp.float32),
          jax.ShapeDtypeStruct((NP, 2), jnp.float32),
      ],
  )


def _tc3(NP, NCLASS, R):
  def body(os_ref, rs_ref, out_ref):
    o = (os_ref[0] + os_ref[1]) / (rs_ref[0] + rs_ref[1])
    m = jnp.max(o, axis=1, keepdims=True)
    z = o - m
    lse = jnp.log(jnp.sum(jnp.exp(z), axis=1, keepdims=True))
    out_ref[...] = z - lse

  return pl.pallas_call(
      body,
      grid=(NP // R,),
      in_specs=[
          pl.BlockSpec((2, R, NCLASS), lambda i: (0, i, 0)),
          pl.BlockSpec((2, R, 1), lambda i: (0, i, 0)),
      ],
      out_specs=pl.BlockSpec((R, NCLASS), lambda i: (i, 0)),
      out_shape=jax.ShapeDtypeStruct((NP, NCLASS), jnp.float32),
  )


def kernel(features, edge_list, W1, b1, a1, W2, b2, a2):
  N, F_IN = features.shape
  E = edge_list.shape[1]
  NH, NHID, _ = W1.shape
  NCLASS = W2.shape[0]
  HD = NH * NHID
  NPAIR = NH // 2

  NP = ((N + NSUB * BLK - 1) // (NSUB * BLK)) * (NSUB * BLK)
  EPT = ((E + NW * BLK * CHB - 1) // (NW * BLK * CHB)) * BLK * CHB
  nblk = EPT // BLK
  EPAD = EPT * NW

  src = edge_list[0]
  dst = edge_list[1]
  pad = jnp.arange(EPAD - E, dtype=jnp.int32) % N
  srcp = jnp.concatenate([src, pad]).reshape(NW, nblk, BLK)
  dstp = jnp.concatenate([dst, pad]).reshape(NW, nblk, BLK)

  xp = jnp.concatenate(
      [features, jnp.zeros((NP - N, F_IN), jnp.float32)], axis=0)

  W1T = W1.reshape(HD, F_IN).T
  b1f = b1.reshape(1, HD)
  al = a1[:, :NHID]
  ar = a1[:, NHID:]
  eye = jnp.eye(NH, dtype=jnp.float32)
  abd_l = (al[:, :, None] * eye[:, None, :]).reshape(HD, NH)
  abd_r = (ar[:, :, None] * eye[:, None, :]).reshape(HD, NH)
  abd = jnp.concatenate([abd_l, abd_r], axis=1)

  h_all, eler = _tc1(NP, F_IN, HD, 2 * NH, 512)(xp, W1T, b1f, abd)
  elT = eler[:, :NH].T
  erT = eler[:, NH:].T
  hpack = h_all.reshape(NP, NPAIR, FW).transpose(1, 0, 2).reshape(
      NPAIR * NP, FW)

  evals, rsum = _sc_attn(NPAIR, 2, NP, E, EPT, nblk)(srcp, dstp, elT, erT)
  rsum = rsum.reshape(NSC, NH, NP)

  hsum = _sc_scatter(NPAIR, 2, NP, EPT, nblk)(srcp, dstp, evals, hpack)

  W2r = W2.reshape(NCLASS, NH, NHID).transpose(1, 2, 0)
  b2f = b2.reshape(1, NCLASS)
  a2lr = jnp.stack([a2[:NCLASS], a2[NCLASS:]], axis=1)
  g, eler2 = _tc2(NP, NH, NHID, NCLASS, 256)(
      hsum, rsum[..., None], W2r, b2f, a2lr)
  el2T = eler2[:, 0].reshape(1, NP)
  er2T = eler2[:, 1].reshape(1, NP)

  osum, rsum2 = _sc_edge_stage2(NP, E, EPT, nblk, NCLASS)(
      srcp, dstp, el2T, er2T, g)
  osum = osum.reshape(NSC, NP, NCLASS)
  rsum2 = rsum2.reshape(NSC, NP)

  out = _tc3(NP, NCLASS, 512)(osum, rsum2[..., None])
  return out[:N]
